# Initial kernel scaffold; baseline (speedup 1.0000x reference)
import jax
import jax.numpy as jnp
from jax.experimental import pallas as pl


def kernel(x, edge_index, batch, W0, b0, g0, be0, W1, b1, g1, be1, W2, b2, g2, be2, W3, b3, g3, be3):
    raise NotImplementedError("write your pallas kernel here")


# speedup vs baseline: 7.6794x; 7.6794x over previous
import functools
import math

import jax
import jax.numpy as jnp
from jax import lax
from jax.experimental import pallas as pl
from jax.experimental.pallas import tpu as pltpu
from jax.experimental.pallas import tpu_sc as plsc

_N = 10000
_E = 320000
_G = 64
_EPS = 1e-5

_NC = 2
_NS = 16
_K = 80

_BLK = 1000
_GRID = _N // _BLK


def _mesh():
  return plsc.VectorSubcoreMesh(core_axis_name="c", subcore_axis_name="s",
                                num_cores=_NC, num_subcores=_NS)


def _rowcopy(s, get_src, get_dst):
  @pl.when(s < _NS - 1)
  def _():
    r0 = pl.multiple_of(s * 632, 8)
    pltpu.sync_copy(get_src().at[pl.ds(r0, 632)],
                    get_dst().at[pl.ds(r0, 632)])

  @pl.when(s == _NS - 1)
  def _():
    pltpu.sync_copy(get_src().at[pl.ds(9480, 520)],
                    get_dst().at[pl.ds(9480, 520)])


def _segsum_body(q, src, dst, out, sidx, didx, rows, acc, sem, s,
                 e_base, n_chunk):
  @pl.loop(0, n_chunk)
  def _(j):
    base = e_base + j * _K
    pltpu.sync_copy(src.at[pl.ds(base, _K)], sidx)
    pltpu.sync_copy(dst.at[pl.ds(base, _K)], didx)
    pltpu.async_copy(q.at[sidx], rows, sem).wait()
    pltpu.sync_copy(rows, acc.at[didx], add=True)

  plsc.subcore_barrier()
  _rowcopy(s, lambda: acc, lambda: out)


def _sc_scratch(C):
  return [
      pltpu.VMEM((_K,), jnp.int32),
      pltpu.VMEM((_K,), jnp.int32),
      pltpu.VMEM((_K, C), jnp.float32),
      pltpu.VMEM_SHARED((_N, C), jnp.float32),
      pltpu.SemaphoreType.DMA,
  ]


@functools.lru_cache(maxsize=None)
def _make_segsum_feat(C):
  ept = _E // _NS
  n_chunk = ept // _K

  @functools.partial(
      pl.kernel,
      out_type=(jax.ShapeDtypeStruct((_N, C), jnp.float32),
                jax.ShapeDtypeStruct((_N, C), jnp.float32)),
      mesh=_mesh(),
      scratch_types=_sc_scratch(C),
  )
  def seg(q_a, q_b, src, dst, out_a, out_b, sidx, didx, rows, acc, sem):
    s = lax.axis_index("s")
    c = lax.axis_index("c")
    e0 = s * ept

    def run(q, out):
      _rowcopy(s, lambda: q, lambda: acc)
      plsc.subcore_barrier()
      _segsum_body(q, src, dst, out, sidx, didx, rows, acc, sem, s,
                   e0, n_chunk)

    @pl.when(c == 0)
    def _():
      run(q_a, out_a)

    @pl.when(c == 1)
    def _():
      run(q_b, out_b)

  return seg


@functools.lru_cache(maxsize=None)
def _make_segsum_edge(C):
  ept = _E // (2 * _NS)
  n_chunk = ept // _K

  @functools.partial(
      pl.kernel,
      out_type=(jax.ShapeDtypeStruct((_N, C), jnp.float32),
                jax.ShapeDtypeStruct((_N, C), jnp.float32)),
      mesh=_mesh(),
      scratch_types=_sc_scratch(C),
  )
  def seg(q, zer, src, dst, out_a, out_b, sidx, didx, rows, acc, sem):
    s = lax.axis_index("s")
    c = lax.axis_index("c")

    def run(init, out, e0):
      _rowcopy(s, lambda: init, lambda: acc)
      plsc.subcore_barrier()
      _segsum_body(q, src, dst, out, sidx, didx, rows, acc, sem, s,
                   e0, n_chunk)

    @pl.when(c == 0)
    def _():
      run(q, out_a, s * ept)

    @pl.when(c == 1)
    def _():
      run(zer, out_b, _E // 2 + s * ept)

  return seg


@functools.lru_cache(maxsize=None)
def _make_deg():
  C = 16
  ept = _E // (2 * _NS)
  n_chunk = ept // _K

  @functools.partial(
      pl.kernel,
      out_type=(jax.ShapeDtypeStruct((_N, C), jnp.float32),
                jax.ShapeDtypeStruct((_N, C), jnp.float32)),
      mesh=_mesh(),
      scratch_types=[
          pltpu.VMEM((_K,), jnp.int32),
          pltpu.VMEM((_K, C), jnp.float32),
          pltpu.VMEM_SHARED((_N, C), jnp.float32),
      ],
  )
  def deg(ones, zer, dst, out_a, out_b, didx, ones_rows, acc):
    s = lax.axis_index("s")
    c = lax.axis_index("c")
    pltpu.sync_copy(ones.at[pl.ds(0, _K)], ones_rows)

    def run(init, out, e0):
      _rowcopy(s, lambda: init, lambda: acc)
      plsc.subcore_barrier()

      @pl.loop(0, n_chunk)
      def _(j):
        base = e0 + j * _K
        pltpu.sync_copy(dst.at[pl.ds(base, _K)], didx)
        pltpu.sync_copy(ones_rows, acc.at[didx], add=True)

      plsc.subcore_barrier()
      _rowcopy(s, lambda: acc, lambda: out)

    @pl.when(c == 0)
    def _():
      run(ones, out_a, s * ept)

    @pl.when(c == 1)
    def _():
      run(zer, out_b, _E // 2 + s * ept)

  return deg


def _row(C):
  return pl.BlockSpec((_BLK, C), lambda k: (k, 0))


def _full(shape):
  return pl.BlockSpec(shape, lambda k: tuple(0 for _ in shape))


def _dot(a, b):
  return jnp.dot(a, b, preferred_element_type=jnp.float32,
                 precision=lax.Precision.HIGHEST)


def _prep_body(x_ref, da_ref, db_ref, w0x_ref, w0c_ref,
               qa_ref, qb_ref, dinv_ref):
  degp1 = da_ref[:, 0:1] + db_ref[:, 0:1]
  dinv = lax.rsqrt(degp1)
  dinv_ref[...] = dinv
  y = _dot(x_ref[...], w0x_ref[...]) + (degp1 - 1.0) * w0c_ref[...]
  qa_ref[...] = y[:, 0:128] * dinv
  qb_ref[...] = y[:, 128:256] * dinv


_prep = pl.pallas_call(
    _prep_body,
    grid=(_GRID,),
    in_specs=[_row(128), _row(16), _row(16), _full((128, 256)),
              _full((1, 256))],
    out_specs=[_row(128), _row(128), _row(1)],
    out_shape=[jax.ShapeDtypeStruct((_N, 128), jnp.float32),
               jax.ShapeDtypeStruct((_N, 128), jnp.float32),
               jax.ShapeDtypeStruct((_N, 1), jnp.float32)],
)


def _bn0_body(sa_ref, sb_ref, dinv_ref, gs_ref, bb_ref, qa_ref, qb_ref):
  dinv = dinv_ref[...]
  ha = jnp.maximum(sa_ref[...] * dinv * gs_ref[:, 0:128]
                   + bb_ref[:, 0:128], 0.0)
  hb = jnp.maximum(sb_ref[...] * dinv * gs_ref[:, 128:256]
                   + bb_ref[:, 128:256], 0.0)
  qa_ref[...] = ha * dinv
  qb_ref[...] = hb * dinv


_bn0 = pl.pallas_call(
    _bn0_body,
    grid=(_GRID,),
    in_specs=[_row(128), _row(128), _row(1), _full((1, 256)),
              _full((1, 256))],
    out_specs=[_row(128), _row(128)],
    out_shape=[jax.ShapeDtypeStruct((_N, 128), jnp.float32),
               jax.ShapeDtypeStruct((_N, 128), jnp.float32)],
)


def _layer1_body(sa_ref, sb_ref, dinv_ref, w_ref, gs_ref, bb_ref,
                 qa_ref, qb_ref):
  dinv = dinv_ref[...]
  out = (_dot(sa_ref[...] * dinv, w_ref[0:128, :]) +
         _dot(sb_ref[...] * dinv, w_ref[128:256, :]))
  h = jnp.maximum(out * gs_ref[...] + bb_ref[...], 0.0)
  qa_ref[...] = h[:, 0:128] * dinv
  qb_ref[...] = h[:, 128:256] * dinv


_layer1 = pl.pallas_call(
    _layer1_body,
    grid=(_GRID,),
    in_specs=[_row(128), _row(128), _row(1), _full((256, 256)),
              _full((1, 256)), _full((1, 256))],
    out_specs=[_row(128), _row(128)],
    out_shape=[jax.ShapeDtypeStruct((_N, 128), jnp.float32),
               jax.ShapeDtypeStruct((_N, 128), jnp.float32)],
)


def _layer2_body(sa_ref, sb_ref, dinv_ref, w2_ref, gs_ref, bb_ref, w3_ref,
                 q3_ref):
  dinv = dinv_ref[...]
  out = (_dot(sa_ref[...] * dinv, w2_ref[0:128, :]) +
         _dot(sb_ref[...] * dinv, w2_ref[128:256, :]))
  h = jnp.maximum(out * gs_ref[...] + bb_ref[...], 0.0)
  q3_ref[...] = _dot(h, w3_ref[...]) * dinv


_layer2 = pl.pallas_call(
    _layer2_body,
    grid=(_GRID,),
    in_specs=[_row(128), _row(128), _row(1), _full((256, 256)),
              _full((1, 256)), _full((1, 256)), _full((256, 128))],
    out_specs=[_row(128)],
    out_shape=[jax.ShapeDtypeStruct((_N, 128), jnp.float32)],
)


def _final_body(sa_ref, sb_ref, dinv_ref, gs_ref, bb_ref, batch_ref,
                node_ref, graph_ref, acc_ref):
  z = (sa_ref[...] + sb_ref[...]) * dinv_ref[...]
  zb = z * gs_ref[...] + bb_ref[...]
  node = jnp.where(zb >= 0, zb, 0.2 * zb)
  node_ref[...] = node
  k = pl.program_id(0)

  @pl.when(k == 0)
  def _():
    acc_ref[...] = jnp.zeros_like(acc_ref)

  onehot = (lax.broadcasted_iota(jnp.int32, (_G, _BLK), 0)
            == batch_ref[0]).astype(jnp.float32)
  acc_ref[...] += _dot(onehot, node)

  @pl.when(k == _GRID - 1)
  def _():
    graph_ref[...] = acc_ref[...]


_final = pl.pallas_call(
    _final_body,
    grid=(_GRID,),
    in_specs=[_row(128), _row(128), _row(1), _full((1, 128)),
              _full((1, 128)), pl.BlockSpec((1, 1, _BLK), lambda k: (k, 0, 0))],
    out_specs=[_row(128), _full((_G, 128))],
    out_shape=[jax.ShapeDtypeStruct((_N, 128), jnp.float32),
               jax.ShapeDtypeStruct((_G, 128), jnp.float32)],
    scratch_shapes=[pltpu.VMEM((_G, 128), jnp.float32)],
)


def _segsum_feat(qa, qb, src, dst):
  return _make_segsum_feat(128)(qa, qb, src, dst)


def _segsum_edge(q, zer, src, dst):
  return _make_segsum_edge(128)(q, zer, src, dst)


def _deg(ones, zer, dst):
  return _make_deg()(ones, zer, dst)


def kernel(x, edge_index, batch, W0, b0, g0, be0, W1, b1, g1, be1,
           W2, b2, g2, be2, W3, b3, g3, be3):
  f32 = jnp.float32
  src = edge_index[0]
  dst = edge_index[1]
  sc = 1.0 / math.sqrt(1.0 + _EPS)

  def bn(g, b, be):
    gs = (g * sc).reshape(1, -1).astype(f32)
    bb = (b * gs[0] + be).reshape(1, -1).astype(f32)
    return gs, bb

  gs0, bb0 = bn(g0, b0, be0)
  gs1, bb1 = bn(g1, b1, be1)
  gs2, bb2 = bn(g2, b2, be2)
  gs3, bb3 = bn(g3, b3, be3)
  w0x = W0[0:128]
  w0c = W0[128:129]
  batch3 = batch.reshape(_GRID, 1, _BLK)

  ones16 = jnp.ones((_N, 16), f32)
  zer16 = jnp.zeros((_N, 16), f32)
  zer128 = jnp.zeros((_N, 128), f32)

  da, db = _deg(ones16, zer16, dst)
  q0a, q0b, dinv = _prep(x, da, db, w0x, w0c)
  s0a, s0b = _segsum_feat(q0a, q0b, src, dst)
  q1a, q1b = _bn0(s0a, s0b, dinv, gs0, bb0)
  s1a, s1b = _segsum_feat(q1a, q1b, src, dst)
  q2a, q2b = _layer1(s1a, s1b, dinv, W1, gs1, bb1)
  s2a, s2b = _segsum_feat(q2a, q2b, src, dst)
  (q3,) = _layer2(s2a, s2b, dinv, W2, gs2, bb2, W3)
  s3a, s3b = _segsum_edge(q3, zer128, src, dst)
  node, graph = _final(s3a, s3b, dinv, gs3, bb3, batch3)
  return node, graph


# speedup vs baseline: 14.7401x; 1.9194x over previous
import functools
import math

import jax
import jax.numpy as jnp
from jax import lax
from jax.experimental import pallas as pl
from jax.experimental.pallas import tpu as pltpu
from jax.experimental.pallas import tpu_sc as plsc

_N = 10000
_E = 320000
_G = 64
_EPS = 1e-5

_NC = 2
_NS = 16
_K = 80

_BLK = 1000
_GRID = _N // _BLK


def _mesh():
  return plsc.VectorSubcoreMesh(core_axis_name="c", subcore_axis_name="s",
                                num_cores=_NC, num_subcores=_NS)


def _rowcopy(s, get_src, get_dst):
  @pl.when(s < _NS - 1)
  def _():
    r0 = pl.multiple_of(s * 632, 8)
    pltpu.sync_copy(get_src().at[pl.ds(r0, 632)],
                    get_dst().at[pl.ds(r0, 632)])

  @pl.when(s == _NS - 1)
  def _():
    pltpu.sync_copy(get_src().at[pl.ds(9480, 520)],
                    get_dst().at[pl.ds(9480, 520)])


def _segsum_body(q, src, dst, out, e0, n_chunk, sml_s, sml_d, rows,
                 isems, gsems, ssems, acc, s):
  def i_fill(j, b):
    ds = pl.ds(e0 + j * _K, _K)
    pltpu.make_async_copy(src.at[ds], sml_s[b], isems[b]).start()
    pltpu.make_async_copy(dst.at[ds], sml_d[b], isems[b]).start()

  def i_wait(j, b):
    ds = pl.ds(e0 + j * _K, _K)
    pltpu.make_async_copy(src.at[ds], sml_s[b], isems[b]).wait()
    pltpu.make_async_copy(dst.at[ds], sml_d[b], isems[b]).wait()

  def g_start(b):
    pltpu.make_async_copy(q.at[sml_s[b]], rows[b], gsems[b]).start()

  def g_wait(b):
    pltpu.make_async_copy(q.at[sml_s[b]], rows[b], gsems[b]).wait()

  def s_start(b):
    pltpu.make_async_copy(rows[b], acc.at[sml_d[b]], ssems[b]).start(add=True)

  def s_wait(b):
    pltpu.make_async_copy(rows[b], acc.at[sml_d[b]], ssems[b]).wait()

  def step(j, b, next_gather, prefetch):
    g_wait(b)
    s_start(b)
    if next_gather:
      i_wait(j + 1, 1 - b)
      g_start(1 - b)
    s_wait(b)
    if prefetch:
      i_fill(j + 2, b)

  i_fill(0, 0)
  i_wait(0, 0)
  g_start(0)
  i_fill(1, 1)

  @pl.loop(0, n_chunk - 2)
  def _(j):
    @pl.when(j % 2 == 0)
    def _():
      step(j, 0, True, True)

    @pl.when(j % 2 == 1)
    def _():
      step(j, 1, True, True)

  step(n_chunk - 2, (n_chunk - 2) % 2, True, False)
  step(n_chunk - 1, (n_chunk - 1) % 2, False, False)

  plsc.subcore_barrier()
  _rowcopy(s, lambda: acc, lambda: out)


def _sc_scratch(C, n_chunk):
  del n_chunk
  return [
      pltpu.VMEM((_K,), jnp.int32),
      pltpu.VMEM((_K,), jnp.int32),
      pltpu.VMEM((_K,), jnp.int32),
      pltpu.VMEM((_K,), jnp.int32),
      pltpu.VMEM((_K, C), jnp.float32),
      pltpu.VMEM((_K, C), jnp.float32),
      pltpu.VMEM_SHARED((_N, C), jnp.float32),
      pltpu.SemaphoreType.DMA,
      pltpu.SemaphoreType.DMA,
      pltpu.SemaphoreType.DMA,
      pltpu.SemaphoreType.DMA,
      pltpu.SemaphoreType.DMA,
      pltpu.SemaphoreType.DMA,
  ]


@functools.lru_cache(maxsize=None)
def _make_segsum_feat(C):
  n_chunk = _E // _NS // _K

  @functools.partial(
      pl.kernel,
      out_type=(jax.ShapeDtypeStruct((_N, C), jnp.float32),
                jax.ShapeDtypeStruct((_N, C), jnp.float32)),
      mesh=_mesh(),
      scratch_types=_sc_scratch(C, n_chunk),
  )
  def seg(q_a, q_b, src, dst, out_a, out_b, sa0, sa1, da0, da1, r0, r1,
          acc, i0, i1, g0, g1, s0, s1):
    s = lax.axis_index("s")
    c = lax.axis_index("c")
    e0 = s * (n_chunk * _K)

    def run(q, out):
      _rowcopy(s, lambda: q, lambda: acc)
      plsc.subcore_barrier()
      _segsum_body(q, src, dst, out, e0, n_chunk, (sa0, sa1), (da0, da1),
                   (r0, r1), (i0, i1), (g0, g1), (s0, s1), acc, s)

    @pl.when(c == 0)
    def _():
      run(q_a, out_a)

    @pl.when(c == 1)
    def _():
      run(q_b, out_b)

  return seg


@functools.lru_cache(maxsize=None)
def _make_segsum_edge(C):
  n_chunk = _E // (2 * _NS) // _K

  @functools.partial(
      pl.kernel,
      out_type=(jax.ShapeDtypeStruct((_N, C), jnp.float32),
                jax.ShapeDtypeStruct((_N, C), jnp.float32)),
      mesh=_mesh(),
      scratch_types=_sc_scratch(C, n_chunk),
  )
  def seg(q, zer, src, dst, out_a, out_b, sa0, sa1, da0, da1, r0, r1,
          acc, i0, i1, g0, g1, s0, s1):
    s = lax.axis_index("s")
    c = lax.axis_index("c")
    e0 = (c * _NS + s) * (n_chunk * _K)

    def run(init, out):
      _rowcopy(s, lambda: init, lambda: acc)
      plsc.subcore_barrier()
      _segsum_body(q, src, dst, out, e0, n_chunk, (sa0, sa1), (da0, da1),
                   (r0, r1), (i0, i1), (g0, g1), (s0, s1), acc, s)

    @pl.when(c == 0)
    def _():
      run(q, out_a)

    @pl.when(c == 1)
    def _():
      run(zer, out_b)

  return seg


@functools.lru_cache(maxsize=None)
def _make_deg():
  C = 16
  n_chunk = _E // (2 * _NS) // _K

  @functools.partial(
      pl.kernel,
      out_type=(jax.ShapeDtypeStruct((_N, C), jnp.float32),
                jax.ShapeDtypeStruct((_N, C), jnp.float32)),
      mesh=_mesh(),
      scratch_types=[
          pltpu.VMEM((n_chunk * _K,), jnp.int32),
          pltpu.VMEM((_K,), jnp.int32),
          pltpu.VMEM((_K, C), jnp.float32),
          pltpu.VMEM_SHARED((_N, C), jnp.float32),
      ],
  )
  def deg(ones, zer, dst, out_a, out_b, didx, dsml, ones_rows, acc):
    s = lax.axis_index("s")
    c = lax.axis_index("c")
    e0 = (c * _NS + s) * (n_chunk * _K)
    pltpu.sync_copy(ones.at[pl.ds(0, _K)], ones_rows)
    pltpu.sync_copy(dst.at[pl.ds(e0, n_chunk * _K)], didx)

    def run(init, out):
      _rowcopy(s, lambda: init, lambda: acc)
      plsc.subcore_barrier()

      @pl.loop(0, n_chunk)
      def _(j):
        for i in range(_K // 16):
          dsml[pl.ds(i * 16, 16)] = didx[pl.ds(j * _K + i * 16, 16)]
        pltpu.sync_copy(ones_rows, acc.at[dsml], add=True)

      plsc.subcore_barrier()
      _rowcopy(s, lambda: acc, lambda: out)

    @pl.when(c == 0)
    def _():
      run(ones, out_a)

    @pl.when(c == 1)
    def _():
      run(zer, out_b)

  return deg


def _row(C):
  return pl.BlockSpec((_BLK, C), lambda k: (k, 0))


def _full(shape):
  return pl.BlockSpec(shape, lambda k: tuple(0 for _ in shape))


def _dot(a, b):
  return jnp.dot(a, b, preferred_element_type=jnp.float32,
                 precision=lax.Precision.HIGHEST)


def _prep_body(x_ref, da_ref, db_ref, w0x_ref, w0c_ref,
               qa_ref, qb_ref, dinv_ref):
  degp1 = da_ref[:, 0:1] + db_ref[:, 0:1]
  dinv = lax.rsqrt(degp1)
  dinv_ref[...] = dinv
  y = _dot(x_ref[...], w0x_ref[...]) + (degp1 - 1.0) * w0c_ref[...]
  qa_ref[...] = y[:, 0:128] * dinv
  qb_ref[...] = y[:, 128:256] * dinv


_prep = pl.pallas_call(
    _prep_body,
    grid=(_GRID,),
    in_specs=[_row(128), _row(16), _row(16), _full((128, 256)),
              _full((1, 256))],
    out_specs=[_row(128), _row(128), _row(1)],
    out_shape=[jax.ShapeDtypeStruct((_N, 128), jnp.float32),
               jax.ShapeDtypeStruct((_N, 128), jnp.float32),
               jax.ShapeDtypeStruct((_N, 1), jnp.float32)],
)


def _bn0_body(sa_ref, sb_ref, dinv_ref, gs_ref, bb_ref, qa_ref, qb_ref):
  dinv = dinv_ref[...]
  ha = jnp.maximum(sa_ref[...] * dinv * gs_ref[:, 0:128]
                   + bb_ref[:, 0:128], 0.0)
  hb = jnp.maximum(sb_ref[...] * dinv * gs_ref[:, 128:256]
                   + bb_ref[:, 128:256], 0.0)
  qa_ref[...] = ha * dinv
  qb_ref[...] = hb * dinv


_bn0 = pl.pallas_call(
    _bn0_body,
    grid=(_GRID,),
    in_specs=[_row(128), _row(128), _row(1), _full((1, 256)),
              _full((1, 256))],
    out_specs=[_row(128), _row(128)],
    out_shape=[jax.ShapeDtypeStruct((_N, 128), jnp.float32),
               jax.ShapeDtypeStruct((_N, 128), jnp.float32)],
)


def _layer1_body(sa_ref, sb_ref, dinv_ref, w_ref, gs_ref, bb_ref,
                 qa_ref, qb_ref):
  dinv = dinv_ref[...]
  out = (_dot(sa_ref[...] * dinv, w_ref[0:128, :]) +
         _dot(sb_ref[...] * dinv, w_ref[128:256, :]))
  h = jnp.maximum(out * gs_ref[...] + bb_ref[...], 0.0)
  qa_ref[...] = h[:, 0:128] * dinv
  qb_ref[...] = h[:, 128:256] * dinv


_layer1 = pl.pallas_call(
    _layer1_body,
    grid=(_GRID,),
    in_specs=[_row(128), _row(128), _row(1), _full((256, 256)),
              _full((1, 256)), _full((1, 256))],
    out_specs=[_row(128), _row(128)],
    out_shape=[jax.ShapeDtypeStruct((_N, 128), jnp.float32),
               jax.ShapeDtypeStruct((_N, 128), jnp.float32)],
)


def _layer2_body(sa_ref, sb_ref, dinv_ref, w2_ref, gs_ref, bb_ref, w3_ref,
                 q3_ref):
  dinv = dinv_ref[...]
  out = (_dot(sa_ref[...] * dinv, w2_ref[0:128, :]) +
         _dot(sb_ref[...] * dinv, w2_ref[128:256, :]))
  h = jnp.maximum(out * gs_ref[...] + bb_ref[...], 0.0)
  q3_ref[...] = _dot(h, w3_ref[...]) * dinv


_layer2 = pl.pallas_call(
    _layer2_body,
    grid=(_GRID,),
    in_specs=[_row(128), _row(128), _row(1), _full((256, 256)),
              _full((1, 256)), _full((1, 256)), _full((256, 128))],
    out_specs=[_row(128)],
    out_shape=[jax.ShapeDtypeStruct((_N, 128), jnp.float32)],
)


def _final_body(sa_ref, sb_ref, dinv_ref, gs_ref, bb_ref, batch_ref,
                node_ref, graph_ref, acc_ref):
  z = (sa_ref[...] + sb_ref[...]) * dinv_ref[...]
  zb = z * gs_ref[...] + bb_ref[...]
  node = jnp.where(zb >= 0, zb, 0.2 * zb)
  node_ref[...] = node
  k = pl.program_id(0)

  @pl.when(k == 0)
  def _():
    acc_ref[...] = jnp.zeros_like(acc_ref)

  onehot = (lax.broadcasted_iota(jnp.int32, (_G, _BLK), 0)
            == batch_ref[0]).astype(jnp.float32)
  acc_ref[...] += _dot(onehot, node)

  @pl.when(k == _GRID - 1)
  def _():
    graph_ref[...] = acc_ref[...]


_final = pl.pallas_call(
    _final_body,
    grid=(_GRID,),
    in_specs=[_row(128), _row(128), _row(1), _full((1, 128)),
              _full((1, 128)), pl.BlockSpec((1, 1, _BLK), lambda k: (k, 0, 0))],
    out_specs=[_row(128), _full((_G, 128))],
    out_shape=[jax.ShapeDtypeStruct((_N, 128), jnp.float32),
               jax.ShapeDtypeStruct((_G, 128), jnp.float32)],
    scratch_shapes=[pltpu.VMEM((_G, 128), jnp.float32)],
)


def _segsum_feat(qa, qb, src, dst):
  return _make_segsum_feat(128)(qa, qb, src, dst)


def _segsum_edge(q, zer, src, dst):
  return _make_segsum_edge(128)(q, zer, src, dst)


def _deg(ones, zer, dst):
  return _make_deg()(ones, zer, dst)


def kernel(x, edge_index, batch, W0, b0, g0, be0, W1, b1, g1, be1,
           W2, b2, g2, be2, W3, b3, g3, be3):
  f32 = jnp.float32
  src = edge_index[0]
  dst = edge_index[1]
  sc = 1.0 / math.sqrt(1.0 + _EPS)

  def bn(g, b, be):
    gs = (g * sc).reshape(1, -1).astype(f32)
    bb = (b * gs[0] + be).reshape(1, -1).astype(f32)
    return gs, bb

  gs0, bb0 = bn(g0, b0, be0)
  gs1, bb1 = bn(g1, b1, be1)
  gs2, bb2 = bn(g2, b2, be2)
  gs3, bb3 = bn(g3, b3, be3)
  w0x = W0[0:128]
  w0c = W0[128:129]
  batch3 = batch.reshape(_GRID, 1, _BLK)

  ones16 = jnp.ones((_N, 16), f32)
  zer16 = jnp.zeros((_N, 16), f32)
  zer128 = jnp.zeros((_N, 128), f32)

  da, db = _deg(ones16, zer16, dst)
  q0a, q0b, dinv = _prep(x, da, db, w0x, w0c)
  s0a, s0b = _segsum_feat(q0a, q0b, src, dst)
  q1a, q1b = _bn0(s0a, s0b, dinv, gs0, bb0)
  s1a, s1b = _segsum_feat(q1a, q1b, src, dst)
  q2a, q2b = _layer1(s1a, s1b, dinv, W1, gs1, bb1)
  s2a, s2b = _segsum_feat(q2a, q2b, src, dst)
  (q3,) = _layer2(s2a, s2b, dinv, W2, gs2, bb2, W3)
  s3a, s3b = _segsum_edge(q3, zer128, src, dst)
  node, graph = _final(s3a, s3b, dinv, gs3, bb3, batch3)
  return node, graph


# speedup vs baseline: 14.7681x; 1.0019x over previous
import functools
import math

import jax
import jax.numpy as jnp
from jax import lax
from jax.experimental import pallas as pl
from jax.experimental.pallas import tpu as pltpu
from jax.experimental.pallas import tpu_sc as plsc

_N = 10000
_E = 320000
_G = 64
_EPS = 1e-5

_NC = 2
_NS = 16
_K = 80

_BLK = 1000
_GRID = _N // _BLK


def _mesh():
  return plsc.VectorSubcoreMesh(core_axis_name="c", subcore_axis_name="s",
                                num_cores=_NC, num_subcores=_NS)


def _rowcopy(s, get_src, get_dst):
  @pl.when(s < _NS - 1)
  def _():
    r0 = pl.multiple_of(s * 632, 8)
    pltpu.sync_copy(get_src().at[pl.ds(r0, 632)],
                    get_dst().at[pl.ds(r0, 632)])

  @pl.when(s == _NS - 1)
  def _():
    pltpu.sync_copy(get_src().at[pl.ds(9480, 520)],
                    get_dst().at[pl.ds(9480, 520)])


def _segsum_body(q, src, dst, out, e0, n_chunk, sml_s, sml_d, rows,
                 isems, gsems, ssems, acc, s):
  def i_fill(j, b):
    ds = pl.ds(e0 + j * _K, _K)
    pltpu.make_async_copy(src.at[ds], sml_s[b], isems[b]).start()
    pltpu.make_async_copy(dst.at[ds], sml_d[b], isems[b]).start()

  def i_wait(j, b):
    ds = pl.ds(e0 + j * _K, _K)
    pltpu.make_async_copy(src.at[ds], sml_s[b], isems[b]).wait()
    pltpu.make_async_copy(dst.at[ds], sml_d[b], isems[b]).wait()

  def g_start(b):
    pltpu.make_async_copy(q.at[sml_s[b]], rows[b], gsems[b]).start()

  def g_wait(b):
    pltpu.make_async_copy(q.at[sml_s[b]], rows[b], gsems[b]).wait()

  def s_start(b):
    pltpu.make_async_copy(rows[b], acc.at[sml_d[b]], ssems[b]).start(add=True)

  def s_wait(b):
    pltpu.make_async_copy(rows[b], acc.at[sml_d[b]], ssems[b]).wait()

  def step(j, p, next_gather, wait_prev, prefetch):
    g_wait(p)
    s_start(p)
    if next_gather:
      i_wait(j + 1, (p + 1) % _D)
      g_start((p + 1) % _D)
    if wait_prev:
      s_wait((p + 2) % _D)
    if prefetch:
      i_fill(j + 2, (p + 2) % _D)

  i_fill(0, 0)
  i_wait(0, 0)
  g_start(0)
  i_fill(1, 1)
  step(0, 0, True, False, True)
  step(1, 1, True, False, True)

  @pl.loop(2, n_chunk - 2)
  def _(j):
    for p in range(_D):
      @pl.when(j % _D == p)
      def _(p=p):
        step(j, p, True, True, True)

  step(n_chunk - 2, (n_chunk - 2) % _D, True, True, False)
  step(n_chunk - 1, (n_chunk - 1) % _D, False, True, False)
  s_wait((n_chunk - 2) % _D)
  s_wait((n_chunk - 1) % _D)

  plsc.subcore_barrier()
  _rowcopy(s, lambda: acc, lambda: out)


_D = 4


def _sc_scratch(C):
  return ([pltpu.VMEM((_K,), jnp.int32) for _ in range(_D)] +
          [pltpu.VMEM((_K,), jnp.int32) for _ in range(_D)] +
          [pltpu.VMEM((_K, C), jnp.float32) for _ in range(_D)] +
          [pltpu.VMEM_SHARED((_N, C), jnp.float32)] +
          [pltpu.SemaphoreType.DMA for _ in range(3 * _D)])


@functools.lru_cache(maxsize=None)
def _make_segsum_feat(C):
  n_chunk = _E // _NS // _K

  @functools.partial(
      pl.kernel,
      out_type=(jax.ShapeDtypeStruct((_N, C), jnp.float32),
                jax.ShapeDtypeStruct((_N, C), jnp.float32)),
      mesh=_mesh(),
      scratch_types=_sc_scratch(C),
  )
  def seg(q_a, q_b, src, dst, out_a, out_b, *scr):
    sml_s, sml_d, rows = scr[0:_D], scr[_D:2 * _D], scr[2 * _D:3 * _D]
    acc = scr[3 * _D]
    isems = scr[3 * _D + 1:4 * _D + 1]
    gsems = scr[4 * _D + 1:5 * _D + 1]
    ssems = scr[5 * _D + 1:6 * _D + 1]
    s = lax.axis_index("s")
    c = lax.axis_index("c")
    e0 = s * (n_chunk * _K)

    def run(q, out):
      _rowcopy(s, lambda: q, lambda: acc)
      plsc.subcore_barrier()
      _segsum_body(q, src, dst, out, e0, n_chunk, sml_s, sml_d,
                   rows, isems, gsems, ssems, acc, s)

    @pl.when(c == 0)
    def _():
      run(q_a, out_a)

    @pl.when(c == 1)
    def _():
      run(q_b, out_b)

  return seg


@functools.lru_cache(maxsize=None)
def _make_segsum_edge(C):
  n_chunk = _E // (2 * _NS) // _K

  @functools.partial(
      pl.kernel,
      out_type=(jax.ShapeDtypeStruct((_N, C), jnp.float32),
                jax.ShapeDtypeStruct((_N, C), jnp.float32)),
      mesh=_mesh(),
      scratch_types=_sc_scratch(C),
  )
  def seg(q, zer, src, dst, out_a, out_b, *scr):
    sml_s, sml_d, rows = scr[0:_D], scr[_D:2 * _D], scr[2 * _D:3 * _D]
    acc = scr[3 * _D]
    isems = scr[3 * _D + 1:4 * _D + 1]
    gsems = scr[4 * _D + 1:5 * _D + 1]
    ssems = scr[5 * _D + 1:6 * _D + 1]
    s = lax.axis_index("s")
    c = lax.axis_index("c")
    e0 = (c * _NS + s) * (n_chunk * _K)

    def run(init, out):
      _rowcopy(s, lambda: init, lambda: acc)
      plsc.subcore_barrier()
      _segsum_body(q, src, dst, out, e0, n_chunk, sml_s, sml_d,
                   rows, isems, gsems, ssems, acc, s)

    @pl.when(c == 0)
    def _():
      run(q, out_a)

    @pl.when(c == 1)
    def _():
      run(zer, out_b)

  return seg


@functools.lru_cache(maxsize=None)
def _make_deg():
  C = 16
  n_chunk = _E // (2 * _NS) // _K

  @functools.partial(
      pl.kernel,
      out_type=(jax.ShapeDtypeStruct((_N, C), jnp.float32),
                jax.ShapeDtypeStruct((_N, C), jnp.float32)),
      mesh=_mesh(),
      scratch_types=[
          pltpu.VMEM((n_chunk * _K,), jnp.int32),
          pltpu.VMEM((_K,), jnp.int32),
          pltpu.VMEM((_K, C), jnp.float32),
          pltpu.VMEM_SHARED((_N, C), jnp.float32),
      ],
  )
  def deg(ones, zer, dst, out_a, out_b, didx, dsml, ones_rows, acc):
    s = lax.axis_index("s")
    c = lax.axis_index("c")
    e0 = (c * _NS + s) * (n_chunk * _K)
    pltpu.sync_copy(ones.at[pl.ds(0, _K)], ones_rows)
    pltpu.sync_copy(dst.at[pl.ds(e0, n_chunk * _K)], didx)

    def run(init, out):
      _rowcopy(s, lambda: init, lambda: acc)
      plsc.subcore_barrier()

      @pl.loop(0, n_chunk)
      def _(j):
        for i in range(_K // 16):
          dsml[pl.ds(i * 16, 16)] = didx[pl.ds(j * _K + i * 16, 16)]
        pltpu.sync_copy(ones_rows, acc.at[dsml], add=True)

      plsc.subcore_barrier()
      _rowcopy(s, lambda: acc, lambda: out)

    @pl.when(c == 0)
    def _():
      run(ones, out_a)

    @pl.when(c == 1)
    def _():
      run(zer, out_b)

  return deg


def _row(C):
  return pl.BlockSpec((_BLK, C), lambda k: (k, 0))


def _full(shape):
  return pl.BlockSpec(shape, lambda k: tuple(0 for _ in shape))


def _dot(a, b):
  return jnp.dot(a, b, preferred_element_type=jnp.float32,
                 precision=lax.Precision.HIGHEST)


def _prep_body(x_ref, da_ref, db_ref, w0x_ref, w0c_ref,
               qa_ref, qb_ref, dinv_ref):
  degp1 = da_ref[:, 0:1] + db_ref[:, 0:1]
  dinv = lax.rsqrt(degp1)
  dinv_ref[...] = dinv
  y = _dot(x_ref[...], w0x_ref[...]) + (degp1 - 1.0) * w0c_ref[...]
  qa_ref[...] = y[:, 0:128] * dinv
  qb_ref[...] = y[:, 128:256] * dinv


_prep = pl.pallas_call(
    _prep_body,
    grid=(_GRID,),
    in_specs=[_row(128), _row(16), _row(16), _full((128, 256)),
              _full((1, 256))],
    out_specs=[_row(128), _row(128), _row(1)],
    out_shape=[jax.ShapeDtypeStruct((_N, 128), jnp.float32),
               jax.ShapeDtypeStruct((_N, 128), jnp.float32),
               jax.ShapeDtypeStruct((_N, 1), jnp.float32)],
)


def _bn0_body(sa_ref, sb_ref, dinv_ref, gs_ref, bb_ref, qa_ref, qb_ref):
  dinv = dinv_ref[...]
  ha = jnp.maximum(sa_ref[...] * dinv * gs_ref[:, 0:128]
                   + bb_ref[:, 0:128], 0.0)
  hb = jnp.maximum(sb_ref[...] * dinv * gs_ref[:, 128:256]
                   + bb_ref[:, 128:256], 0.0)
  qa_ref[...] = ha * dinv
  qb_ref[...] = hb * dinv


_bn0 = pl.pallas_call(
    _bn0_body,
    grid=(_GRID,),
    in_specs=[_row(128), _row(128), _row(1), _full((1, 256)),
              _full((1, 256))],
    out_specs=[_row(128), _row(128)],
    out_shape=[jax.ShapeDtypeStruct((_N, 128), jnp.float32),
               jax.ShapeDtypeStruct((_N, 128), jnp.float32)],
)


def _layer1_body(sa_ref, sb_ref, dinv_ref, w_ref, gs_ref, bb_ref,
                 qa_ref, qb_ref):
  dinv = dinv_ref[...]
  out = (_dot(sa_ref[...] * dinv, w_ref[0:128, :]) +
         _dot(sb_ref[...] * dinv, w_ref[128:256, :]))
  h = jnp.maximum(out * gs_ref[...] + bb_ref[...], 0.0)
  qa_ref[...] = h[:, 0:128] * dinv
  qb_ref[...] = h[:, 128:256] * dinv


_layer1 = pl.pallas_call(
    _layer1_body,
    grid=(_GRID,),
    in_specs=[_row(128), _row(128), _row(1), _full((256, 256)),
              _full((1, 256)), _full((1, 256))],
    out_specs=[_row(128), _row(128)],
    out_shape=[jax.ShapeDtypeStruct((_N, 128), jnp.float32),
               jax.ShapeDtypeStruct((_N, 128), jnp.float32)],
)


def _layer2_body(sa_ref, sb_ref, dinv_ref, w2_ref, gs_ref, bb_ref, w3_ref,
                 q3_ref):
  dinv = dinv_ref[...]
  out = (_dot(sa_ref[...] * dinv, w2_ref[0:128, :]) +
         _dot(sb_ref[...] * dinv, w2_ref[128:256, :]))
  h = jnp.maximum(out * gs_ref[...] + bb_ref[...], 0.0)
  q3_ref[...] = _dot(h, w3_ref[...]) * dinv


_layer2 = pl.pallas_call(
    _layer2_body,
    grid=(_GRID,),
    in_specs=[_row(128), _row(128), _row(1), _full((256, 256)),
              _full((1, 256)), _full((1, 256)), _full((256, 128))],
    out_specs=[_row(128)],
    out_shape=[jax.ShapeDtypeStruct((_N, 128), jnp.float32)],
)


def _final_body(sa_ref, sb_ref, dinv_ref, gs_ref, bb_ref, batch_ref,
                node_ref, graph_ref, acc_ref):
  z = (sa_ref[...] + sb_ref[...]) * dinv_ref[...]
  zb = z * gs_ref[...] + bb_ref[...]
  node = jnp.where(zb >= 0, zb, 0.2 * zb)
  node_ref[...] = node
  k = pl.program_id(0)

  @pl.when(k == 0)
  def _():
    acc_ref[...] = jnp.zeros_like(acc_ref)

  onehot = (lax.broadcasted_iota(jnp.int32, (_G, _BLK), 0)
            == batch_ref[0]).astype(jnp.float32)
  acc_ref[...] += _dot(onehot, node)

  @pl.when(k == _GRID - 1)
  def _():
    graph_ref[...] = acc_ref[...]


_final = pl.pallas_call(
    _final_body,
    grid=(_GRID,),
    in_specs=[_row(128), _row(128), _row(1), _full((1, 128)),
              _full((1, 128)), pl.BlockSpec((1, 1, _BLK), lambda k: (k, 0, 0))],
    out_specs=[_row(128), _full((_G, 128))],
    out_shape=[jax.ShapeDtypeStruct((_N, 128), jnp.float32),
               jax.ShapeDtypeStruct((_G, 128), jnp.float32)],
    scratch_shapes=[pltpu.VMEM((_G, 128), jnp.float32)],
)


def _segsum_feat(qa, qb, src, dst):
  return _make_segsum_feat(128)(qa, qb, src, dst)


def _segsum_edge(q, zer, src, dst):
  return _make_segsum_edge(128)(q, zer, src, dst)


def _deg(ones, zer, dst):
  return _make_deg()(ones, zer, dst)


def kernel(x, edge_index, batch, W0, b0, g0, be0, W1, b1, g1, be1,
           W2, b2, g2, be2, W3, b3, g3, be3):
  f32 = jnp.float32
  src = edge_index[0]
  dst = edge_index[1]
  sc = 1.0 / math.sqrt(1.0 + _EPS)

  def bn(g, b, be):
    gs = (g * sc).reshape(1, -1).astype(f32)
    bb = (b * gs[0] + be).reshape(1, -1).astype(f32)
    return gs, bb

  gs0, bb0 = bn(g0, b0, be0)
  gs1, bb1 = bn(g1, b1, be1)
  gs2, bb2 = bn(g2, b2, be2)
  gs3, bb3 = bn(g3, b3, be3)
  w0x = W0[0:128]
  w0c = W0[128:129]
  batch3 = batch.reshape(_GRID, 1, _BLK)

  ones16 = jnp.ones((_N, 16), f32)
  zer16 = jnp.zeros((_N, 16), f32)
  zer128 = jnp.zeros((_N, 128), f32)

  da, db = _deg(ones16, zer16, dst)
  q0a, q0b, dinv = _prep(x, da, db, w0x, w0c)
  s0a, s0b = _segsum_feat(q0a, q0b, src, dst)
  q1a, q1b = _bn0(s0a, s0b, dinv, gs0, bb0)
  s1a, s1b = _segsum_feat(q1a, q1b, src, dst)
  q2a, q2b = _layer1(s1a, s1b, dinv, W1, gs1, bb1)
  s2a, s2b = _segsum_feat(q2a, q2b, src, dst)
  (q3,) = _layer2(s2a, s2b, dinv, W2, gs2, bb2, W3)
  s3a, s3b = _segsum_edge(q3, zer128, src, dst)
  node, graph = _final(s3a, s3b, dinv, gs3, bb3, batch3)
  return node, graph
